# baseline (speedup 1.0000x reference)
import functools

import jax
import jax.numpy as jnp
from jax import lax
from jax.experimental import pallas as pl
from jax.experimental.pallas import tpu as pltpu
from jax.experimental.pallas import tpu_sc as plsc

TOKENS = 16384
D_MODEL = 4096
N_EXPERTS = 64
BLK = 1024

NC, NS, L = 2, 16, 16
NW = NC * NS

CHUNKS = 2
CT = TOKENS // CHUNKS
TPW = CT // NW
GROUPS = TPW // L


def _matmul_block(x_ref, w_ref, s_ref):
    s_ref[...] = jax.lax.dot_general(
        w_ref[...], x_ref[...], (((1,), (1,)), ((), ())),
        preferred_element_type=jnp.float32,
        precision=jax.lax.Precision.DEFAULT,
    )


def _scores_t_chunk(x, W, c):
    nblk = CT // BLK
    return pl.pallas_call(
        _matmul_block,
        grid=(nblk,),
        in_specs=[
            pl.BlockSpec((BLK, D_MODEL), lambda i, c=c: (c * nblk + i, 0)),
            pl.BlockSpec((N_EXPERTS, D_MODEL), lambda i: (0, 0)),
        ],
        out_specs=pl.BlockSpec((N_EXPERTS, BLK), lambda i: (0, i)),
        out_shape=jax.ShapeDtypeStruct((N_EXPERTS, CT), jnp.float32),
    )(x, W)


def _sc_router_body(scores_ref, topi_ref, topv_ref, s_v, i_v, v_v):
    wid = lax.axis_index("s") * NC + lax.axis_index("c")
    base = wid * TPW
    pltpu.sync_copy(scores_ref.at[:, pl.ds(base, TPW)], s_v)

    def group(g, _):
        t0 = g * L
        m1 = s_v[0, pl.ds(t0, L)]
        i1 = jnp.zeros((L,), jnp.int32)
        m2 = jnp.full((L,), -3.4e38, jnp.float32)
        i2 = jnp.zeros((L,), jnp.int32)

        for e in range(1, N_EXPERTS):
            v = s_v[e, pl.ds(t0, L)]
            es = jnp.full((L,), e, jnp.int32)
            gt1 = v > m1
            gt2 = v > m2
            i2 = jnp.where(gt1, i1, jnp.where(gt2, es, i2))
            m2 = jnp.where(gt1, m1, jnp.where(gt2, v, m2))
            i1 = jnp.where(gt1, es, i1)
            m1 = jnp.where(gt1, v, m1)

        z = jnp.zeros((L,), jnp.float32)
        for e in range(N_EXPERTS):
            v = s_v[e, pl.ds(t0, L)]
            z = z + jnp.exp(v - m1)

        p1 = 1.0 / z
        p2 = jnp.exp(m2 - m1) / z
        denom = p1 + p2 + 1e-9
        i_v[0, pl.ds(t0, L)] = i1
        i_v[1, pl.ds(t0, L)] = i2
        v_v[0, pl.ds(t0, L)] = p1 / denom
        v_v[1, pl.ds(t0, L)] = p2 / denom
        return 0

    lax.fori_loop(0, GROUPS, group, 0)
    pltpu.sync_copy(i_v, topi_ref.at[:, pl.ds(base, TPW)])
    pltpu.sync_copy(v_v, topv_ref.at[:, pl.ds(base, TPW)])


_sc_router = functools.partial(
    pl.kernel,
    out_type=[
        jax.ShapeDtypeStruct((2, CT), jnp.int32),
        jax.ShapeDtypeStruct((2, CT), jnp.float32),
    ],
    mesh=plsc.VectorSubcoreMesh(core_axis_name="c", subcore_axis_name="s"),
    scratch_types=[
        pltpu.VMEM((N_EXPERTS, TPW), jnp.float32),
        pltpu.VMEM((2, TPW), jnp.int32),
        pltpu.VMEM((2, TPW), jnp.float32),
    ],
)(_sc_router_body)


@jax.jit
def kernel(x, W):
    tis, tvs = [], []
    for c in range(CHUNKS):
        scores_t = _scores_t_chunk(x, W, c)
        ti, tv = _sc_router(scores_t)
        tis.append(ti)
        tvs.append(tv)
    topi_t = jnp.concatenate(tis, axis=1)
    topv_t = jnp.concatenate(tvs, axis=1)
    return topi_t.T, topv_t.T


# speedup vs baseline: 1.0752x; 1.0752x over previous
import functools

import jax
import jax.numpy as jnp
from jax import lax
from jax.experimental import pallas as pl
from jax.experimental.pallas import tpu as pltpu
from jax.experimental.pallas import tpu_sc as plsc

TOKENS = 16384
D_MODEL = 4096
N_EXPERTS = 64
BLK = 1024

NC, NS, L = 2, 16, 16
NW = NC * NS

CHUNKS = 1
CT = TOKENS // CHUNKS
TPW = CT // NW
GROUPS = TPW // L


def _matmul_block(x_ref, w_ref, s_ref):
    s_ref[...] = jax.lax.dot_general(
        w_ref[...], x_ref[...], (((1,), (1,)), ((), ())),
        preferred_element_type=jnp.float32,
        precision=jax.lax.Precision.DEFAULT,
    )


def _scores_t_chunk(x, W, c):
    nblk = CT // BLK
    return pl.pallas_call(
        _matmul_block,
        grid=(nblk,),
        in_specs=[
            pl.BlockSpec((BLK, D_MODEL), lambda i, c=c: (c * nblk + i, 0)),
            pl.BlockSpec((N_EXPERTS, D_MODEL), lambda i: (0, 0)),
        ],
        out_specs=pl.BlockSpec((N_EXPERTS, BLK), lambda i: (0, i)),
        out_shape=jax.ShapeDtypeStruct((N_EXPERTS, CT), jnp.float32),
    )(x, W)


def _sc_router_body(scores_ref, topi_ref, topv_ref, s_v, i_v, v_v):
    wid = lax.axis_index("s") * NC + lax.axis_index("c")
    base = wid * TPW
    pltpu.sync_copy(scores_ref.at[:, pl.ds(base, TPW)], s_v)

    def group(g, _):
        t0 = g * L
        m1 = s_v[0, pl.ds(t0, L)]
        i1 = jnp.zeros((L,), jnp.int32)
        m2 = jnp.full((L,), -3.4e38, jnp.float32)
        i2 = jnp.zeros((L,), jnp.int32)

        for e in range(1, N_EXPERTS):
            v = s_v[e, pl.ds(t0, L)]
            es = jnp.full((L,), e, jnp.int32)
            gt1 = v > m1
            gt2 = v > m2
            i2 = jnp.where(gt1, i1, jnp.where(gt2, es, i2))
            m2 = jnp.where(gt1, m1, jnp.where(gt2, v, m2))
            i1 = jnp.where(gt1, es, i1)
            m1 = jnp.where(gt1, v, m1)

        p1 = jnp.ones((L,), jnp.float32)
        p2 = jnp.exp(m2 - m1)
        denom = p1 + p2 + 1e-9
        i_v[0, pl.ds(t0, L)] = i1
        i_v[1, pl.ds(t0, L)] = i2
        v_v[0, pl.ds(t0, L)] = p1 / denom
        v_v[1, pl.ds(t0, L)] = p2 / denom
        return 0

    lax.fori_loop(0, GROUPS, group, 0)
    pltpu.sync_copy(i_v, topi_ref.at[:, pl.ds(base, TPW)])
    pltpu.sync_copy(v_v, topv_ref.at[:, pl.ds(base, TPW)])


_sc_router = functools.partial(
    pl.kernel,
    out_type=[
        jax.ShapeDtypeStruct((2, CT), jnp.int32),
        jax.ShapeDtypeStruct((2, CT), jnp.float32),
    ],
    mesh=plsc.VectorSubcoreMesh(core_axis_name="c", subcore_axis_name="s"),
    scratch_types=[
        pltpu.VMEM((N_EXPERTS, TPW), jnp.float32),
        pltpu.VMEM((2, TPW), jnp.int32),
        pltpu.VMEM((2, TPW), jnp.float32),
    ],
)(_sc_router_body)


@jax.jit
def kernel(x, W):
    tis, tvs = [], []
    for c in range(CHUNKS):
        scores_t = _scores_t_chunk(x, W, c)
        ti, tv = _sc_router(scores_t)
        tis.append(ti)
        tvs.append(tv)
    topi_t = jnp.concatenate(tis, axis=1)
    topv_t = jnp.concatenate(tvs, axis=1)
    return topi_t.T, topv_t.T
